# baseline (speedup 1.0000x reference)
import functools

import jax
import jax.numpy as jnp
from jax import lax
from jax.experimental import pallas as pl
from jax.experimental.pallas import tpu as pltpu
from jax.experimental.pallas import tpu_sc as plsc

N = 65536
D = 256
NW = 32
CPW = D // NW
WIN = 2048
NVW = WIN // 16
NWINS = N // WIN
MININT = jnp.int32(-(2 ** 31))


def _weights(n):
    grid = jnp.arange(1, n + 1, dtype=jnp.float32)
    pi = (grid - jnp.pi / 8.0) / (n + 0.25)
    m = jax.scipy.stats.norm.ppf(pi)
    return m / jnp.linalg.norm(m)


def _to_key(v):
    u = lax.bitcast_convert_type(v, jnp.int32)
    m = lax.shift_right_arithmetic(u, 31)
    return u ^ (m | MININT)


def _from_key(kk):
    top = lax.shift_right_logical(kk, 31)
    sel = -top
    msk = MININT | (~sel)
    return lax.bitcast_convert_type(kk ^ msk, jnp.float32)


@functools.partial(
    pl.kernel,
    mesh=plsc.VectorSubcoreMesh(core_axis_name="c", subcore_axis_name="s"),
    out_type=[
        jax.ShapeDtypeStruct((D, 16), jnp.float32),
        jax.ShapeDtypeStruct((D, 16), jnp.float32),
        jax.ShapeDtypeStruct((D, N), jnp.int32),
        jax.ShapeDtypeStruct((D, N), jnp.int32),
    ],
    scratch_types=[
        pltpu.VMEM((N,), jnp.int32),
        pltpu.VMEM((2048,), jnp.int32),
        pltpu.VMEM((WIN,), jnp.float32),
        pltpu.VMEM((WIN,), jnp.int32),
        pltpu.VMEM((WIN,), jnp.float32),
        pltpu.VMEM((16,), jnp.float32),
        pltpu.VMEM((16,), jnp.float32),
    ],
    compiler_params=pltpu.CompilerParams(needs_layout_passes=False),
)
def _sw_sc(xT, kvec, num_out, ss_out, s1, s2,
           dest, hist, stage_f, stage_i, stage_k, row_num, row_ss):
    wid = lax.axis_index("c") * 16 + lax.axis_index("s")
    ones_i = jnp.ones((16,), jnp.int32)

    def zero_hist(nbins):
        def zb(i, _):
            hist[pl.ds(i * 16, 16)] = jnp.zeros((16,), jnp.int32)
            return 0
        lax.fori_loop(0, nbins // 16, zb, 0)

    def excl_prefix(nbins):
        def pb(i, carry):
            h = hist[pl.ds(i * 16, 16)]
            inc = plsc.cumsum(h)
            hist[pl.ds(i * 16, 16)] = inc - h + carry
            return carry + jnp.sum(h)
        lax.fori_loop(0, nbins // 16, pb, jnp.int32(0))

    def hist_pass(c, src, from_f32, shift, bmask):
        def wb(w, _):
            if from_f32:
                pltpu.sync_copy(src.at[c, pl.ds(w * WIN, WIN)], stage_f)
            else:
                pltpu.sync_copy(src.at[c, pl.ds(w * WIN, WIN)], stage_i)

            def vb(j, _):
                if from_f32:
                    kk = _to_key(stage_f[pl.ds(j * 16, 16)])
                else:
                    kk = stage_i[pl.ds(j * 16, 16)]
                d = lax.shift_right_logical(kk, shift) & bmask
                plsc.addupdate_scatter(hist, [d], ones_i)
                return 0
            lax.fori_loop(0, NVW, vb, 0)
            return 0
        lax.fori_loop(0, NWINS, wb, 0)

    def permute_pass(c, src, from_f32, shift, bmask):
        def wb(w, _):
            if from_f32:
                pltpu.sync_copy(src.at[c, pl.ds(w * WIN, WIN)], stage_f)
            else:
                pltpu.sync_copy(src.at[c, pl.ds(w * WIN, WIN)], stage_i)

            def vb(j, _):
                if from_f32:
                    kk = _to_key(stage_f[pl.ds(j * 16, 16)])
                else:
                    kk = stage_i[pl.ds(j * 16, 16)]
                d = lax.shift_right_logical(kk, shift) & bmask
                cnt, last = plsc.scan_count(d)
                ofs = plsc.load_gather(hist, [d])
                pos = ofs + cnt - 1
                plsc.store_scatter(dest, [pos], kk)
                plsc.addupdate_scatter(hist, [d], cnt, mask=last)
                return 0
            lax.fori_loop(0, NVW, vb, 0)
            return 0
        lax.fori_loop(0, NWINS, wb, 0)

    def radix_pass(c, src, from_f32, shift, bmask, nbins):
        zero_hist(nbins)
        hist_pass(c, src, from_f32, shift, bmask)
        excl_prefix(nbins)
        permute_pass(c, src, from_f32, shift, bmask)

    def dot_pass(c):
        zf = jnp.zeros((16,), jnp.float32)

        def wb(w, accs):
            pltpu.sync_copy(kvec.at[pl.ds(w * WIN, WIN)], stage_k)

            def vb(j, accs2):
                na, sa = accs2
                kk = dest[pl.ds(w * WIN + j * 16, 16)]
                v = _from_key(kk)
                kv = stage_k[pl.ds(j * 16, 16)]
                return (na + kv * v, sa + v * v)
            return lax.fori_loop(0, NVW, vb, accs)
        na, sa = lax.fori_loop(0, NWINS, wb, (zf, zf))
        row_num[...] = jnp.full((16,), 0.0, jnp.float32) + jnp.sum(na)
        row_ss[...] = jnp.full((16,), 0.0, jnp.float32) + jnp.sum(sa)
        pltpu.sync_copy(row_num, num_out.at[c])
        pltpu.sync_copy(row_ss, ss_out.at[c])

    def col_body(ci, _):
        c = wid * CPW + ci
        radix_pass(c, xT, True, 0, jnp.int32(2047), 2048)
        pltpu.sync_copy(dest, s1.at[c])
        radix_pass(c, s1, False, 11, jnp.int32(2047), 2048)
        pltpu.sync_copy(dest, s2.at[c])
        radix_pass(c, s2, False, 22, jnp.int32(1023), 1024)
        dot_pass(c)
        return 0

    lax.fori_loop(0, CPW, col_body, 0)


def kernel(x):
    eps = 1e-05
    n, d = x.shape
    k = lax.stop_gradient(_weights(n).astype(x.dtype))
    k_norm = jnp.linalg.norm(k)
    xT = x.T
    num_rows, ss_rows, _, _ = _sw_sc(xT, k)
    num = num_rows[:, 0]
    ss = ss_rows[:, 0]
    s_norm = jnp.sqrt(ss)
    cos = num / jnp.maximum(k_norm * s_norm, eps)
    return 1.0 - jnp.abs(cos)


# speedup vs baseline: 1.3692x; 1.3692x over previous
import functools

import jax
import jax.numpy as jnp
from jax import lax
from jax.experimental import pallas as pl
from jax.experimental.pallas import tpu as pltpu
from jax.experimental.pallas import tpu_sc as plsc

N = 65536
D = 256
NW = 32
CPW = D // NW
WIN = 4096
NVW = WIN // 16
NWINS = N // WIN
MININT = jnp.int32(-(2 ** 31))


def _weights(n):
    grid = jnp.arange(1, n + 1, dtype=jnp.float32)
    pi = (grid - jnp.pi / 8.0) / (n + 0.25)
    m = jax.scipy.stats.norm.ppf(pi)
    return m / jnp.linalg.norm(m)


def _to_key(v):
    u = lax.bitcast_convert_type(v, jnp.int32)
    m = lax.shift_right_arithmetic(u, 31)
    return u ^ (m | MININT)


def _from_key(kk):
    top = lax.shift_right_logical(kk, 31)
    sel = -top
    msk = MININT | (~sel)
    return lax.bitcast_convert_type(kk ^ msk, jnp.float32)


@functools.partial(
    pl.kernel,
    mesh=plsc.VectorSubcoreMesh(core_axis_name="c", subcore_axis_name="s"),
    out_type=[
        jax.ShapeDtypeStruct((D, 16), jnp.float32),
        jax.ShapeDtypeStruct((D, 16), jnp.float32),
        jax.ShapeDtypeStruct((D, N), jnp.int32),
        jax.ShapeDtypeStruct((D, N), jnp.int32),
    ],
    scratch_types=[
        pltpu.VMEM((N,), jnp.int32),
        pltpu.VMEM((2048,), jnp.int32),
        pltpu.VMEM((WIN,), jnp.float32),
        pltpu.VMEM((WIN,), jnp.int32),
        pltpu.VMEM((WIN,), jnp.float32),
        pltpu.VMEM((16,), jnp.float32),
        pltpu.VMEM((16,), jnp.float32),
    ],
    compiler_params=pltpu.CompilerParams(needs_layout_passes=False),
)
def _sw_sc(xT, kvec, num_out, ss_out, s1, s2,
           dest, hist, stage_f, stage_i, stage_k, row_num, row_ss):
    wid = lax.axis_index("c") * 16 + lax.axis_index("s")
    ones_i = jnp.ones((16,), jnp.int32)

    def zero_hist(nbins):
        def zb(i, _):
            hist[pl.ds(i * 16, 16)] = jnp.zeros((16,), jnp.int32)
            return 0
        lax.fori_loop(0, nbins // 16, zb, 0)

    def excl_prefix(nbins):
        def pb(i, carry):
            h = hist[pl.ds(i * 16, 16)]
            inc = plsc.cumsum(h)
            hist[pl.ds(i * 16, 16)] = inc - h + carry
            return carry + jnp.sum(h)
        lax.fori_loop(0, nbins // 16, pb, jnp.int32(0))

    def hist_pass(c, src, from_f32, shift, bmask):
        def wb(w, _):
            if from_f32:
                pltpu.sync_copy(src.at[c, pl.ds(w * WIN, WIN)], stage_f)
            else:
                pltpu.sync_copy(src.at[c, pl.ds(w * WIN, WIN)], stage_i)

            @plsc.parallel_loop(0, NVW, step=1, unroll=4)
            def vb(j):
                if from_f32:
                    kk = _to_key(stage_f[pl.ds(j * 16, 16)])
                else:
                    kk = stage_i[pl.ds(j * 16, 16)]
                d = lax.shift_right_logical(kk, shift) & bmask
                plsc.addupdate_scatter(hist, [d], ones_i)
            return 0
        lax.fori_loop(0, NWINS, wb, 0)

    def permute_pass(c, src, from_f32, shift, bmask):
        def wb(w, _):
            if from_f32:
                pltpu.sync_copy(src.at[c, pl.ds(w * WIN, WIN)], stage_f)
            else:
                pltpu.sync_copy(src.at[c, pl.ds(w * WIN, WIN)], stage_i)

            def vb(j, _):
                if from_f32:
                    kk = _to_key(stage_f[pl.ds(j * 16, 16)])
                else:
                    kk = stage_i[pl.ds(j * 16, 16)]
                d = lax.shift_right_logical(kk, shift) & bmask
                cnt, last = plsc.scan_count(d)
                ofs = plsc.load_gather(hist, [d])
                pos = ofs + cnt - 1
                plsc.store_scatter(dest, [pos], kk)
                plsc.addupdate_scatter(hist, [d], cnt, mask=last)
                return 0
            lax.fori_loop(0, NVW, vb, 0, unroll=4)
            return 0
        lax.fori_loop(0, NWINS, wb, 0)

    def radix_pass(c, src, from_f32, shift, bmask, nbins):
        zero_hist(nbins)
        hist_pass(c, src, from_f32, shift, bmask)
        excl_prefix(nbins)
        permute_pass(c, src, from_f32, shift, bmask)

    def dot_pass(c):
        zf = jnp.zeros((16,), jnp.float32)

        def wb(w, accs):
            pltpu.sync_copy(kvec.at[pl.ds(w * WIN, WIN)], stage_k)

            @plsc.parallel_loop(0, NVW, step=1, unroll=4, carry=accs)
            def vb(j, accs2):
                na, ca, sa, cb = accs2
                kk = dest[pl.ds(w * WIN + j * 16, 16)]
                v = _from_key(kk)
                kv = stage_k[pl.ds(j * 16, 16)]
                y1 = kv * v - ca
                t1 = na + y1
                ca = (t1 - na) - y1
                y2 = v * v - cb
                t2 = sa + y2
                cb = (t2 - sa) - y2
                return (t1, ca, t2, cb)
            return vb
        na, _, sa, _ = lax.fori_loop(0, NWINS, wb, (zf, zf, zf, zf))
        row_num[...] = jnp.full((16,), 0.0, jnp.float32) + jnp.sum(na)
        row_ss[...] = jnp.full((16,), 0.0, jnp.float32) + jnp.sum(sa)
        pltpu.sync_copy(row_num, num_out.at[c])
        pltpu.sync_copy(row_ss, ss_out.at[c])

    def col_body(ci, _):
        c = wid * CPW + ci
        radix_pass(c, xT, True, 0, jnp.int32(2047), 2048)
        pltpu.sync_copy(dest, s1.at[c])
        radix_pass(c, s1, False, 11, jnp.int32(2047), 2048)
        pltpu.sync_copy(dest, s2.at[c])
        radix_pass(c, s2, False, 22, jnp.int32(1023), 1024)
        dot_pass(c)
        return 0

    lax.fori_loop(0, CPW, col_body, 0)


def kernel(x):
    eps = 1e-05
    n, d = x.shape
    k = lax.stop_gradient(_weights(n).astype(x.dtype))
    k_norm = jnp.linalg.norm(k)
    xT = x.T
    num_rows, ss_rows, _, _ = _sw_sc(xT, k)
    num = num_rows[:, 0]
    ss = ss_rows[:, 0]
    s_norm = jnp.sqrt(ss)
    cos = num / jnp.maximum(k_norm * s_norm, eps)
    return 1.0 - jnp.abs(cos)


# speedup vs baseline: 1.8103x; 1.3221x over previous
import functools

import jax
import jax.numpy as jnp
from jax import lax
from jax.experimental import pallas as pl
from jax.experimental.pallas import tpu as pltpu
from jax.experimental.pallas import tpu_sc as plsc

N = 65536
D = 256
NW = 32
CPW = D // NW
WIN = 4096
NVW = WIN // 16
NWINS = N // WIN
MININT = jnp.int32(-(2 ** 31))


def _weights(n):
    grid = jnp.arange(1, n + 1, dtype=jnp.float32)
    pi = (grid - jnp.pi / 8.0) / (n + 0.25)
    m = jax.scipy.stats.norm.ppf(pi)
    return m / jnp.linalg.norm(m)


def _to_key(v):
    u = lax.bitcast_convert_type(v, jnp.int32)
    m = lax.shift_right_arithmetic(u, 31)
    return u ^ (m | MININT)


def _from_key(kk):
    top = lax.shift_right_logical(kk, 31)
    msk = MININT | (~(-top))
    return lax.bitcast_convert_type(kk ^ msk, jnp.float32)


@functools.partial(
    pl.kernel,
    mesh=plsc.VectorSubcoreMesh(core_axis_name="c", subcore_axis_name="s"),
    out_type=[
        jax.ShapeDtypeStruct((D, 16), jnp.float32),
        jax.ShapeDtypeStruct((D, 16), jnp.float32),
        jax.ShapeDtypeStruct((D, N), jnp.int32),
        jax.ShapeDtypeStruct((D, N), jnp.int32),
    ],
    scratch_types=[
        pltpu.VMEM((N,), jnp.int32),
        pltpu.VMEM((2048,), jnp.int32),
        pltpu.VMEM((2048,), jnp.int32),
        pltpu.VMEM((1024,), jnp.int32),
        pltpu.VMEM((WIN,), jnp.float32),
        pltpu.VMEM((WIN,), jnp.float32),
        pltpu.VMEM((WIN,), jnp.int32),
        pltpu.VMEM((WIN,), jnp.int32),
        pltpu.VMEM((WIN,), jnp.float32),
        pltpu.VMEM((WIN,), jnp.float32),
        pltpu.VMEM((16,), jnp.float32),
        pltpu.VMEM((16,), jnp.float32),
        pltpu.SemaphoreType.DMA,
        pltpu.SemaphoreType.DMA,
    ],
    compiler_params=pltpu.CompilerParams(needs_layout_passes=False),
)
def _sw_sc(xT, kvec, num_out, ss_out, s1, s2,
           dest, histA, histB, histC,
           sf0, sf1, si0, si1, sk0, sk1, row_num, row_ss, sem0, sem1):
    wid = lax.axis_index("c") * 16 + lax.axis_index("s")
    ones_i = jnp.ones((16,), jnp.int32)
    sems = (sem0, sem1)

    def streamed(src_slice, stages, body, carry_init):
        pltpu.async_copy(src_slice(0), stages[0], sems[0])

        def wb(w2, carry):
            for b in (0, 1):
                w = w2 * 2 + b
                nb = 1 - b

                @pl.when(w + 1 < NWINS)
                def _():
                    pltpu.async_copy(src_slice(w + 1), stages[nb], sems[nb])

                pltpu.make_async_copy(src_slice(w), stages[b], sems[b]).wait()
                carry = body(w, stages[b], carry)
            return carry
        return lax.fori_loop(0, NWINS // 2, wb, carry_init)

    def zero_hist(h, nbins):
        def zb(i, _):
            h[pl.ds(i * 16, 16)] = jnp.zeros((16,), jnp.int32)
            return 0
        lax.fori_loop(0, nbins // 16, zb, 0, unroll=4)

    def excl_prefix(h, nbins):
        def pb(i, carry):
            v = h[pl.ds(i * 16, 16)]
            inc = plsc.cumsum(v)
            h[pl.ds(i * 16, 16)] = inc - v + carry
            return carry + jnp.sum(v)
        lax.fori_loop(0, nbins // 16, pb, jnp.int32(0))

    def hist_all_body(w, stage, carry):
        @plsc.parallel_loop(0, NVW, step=1, unroll=4)
        def vb(j):
            kk = _to_key(stage[pl.ds(j * 16, 16)])
            plsc.addupdate_scatter(histA, [kk & jnp.int32(2047)], ones_i)
            plsc.addupdate_scatter(
                histB, [lax.shift_right_logical(kk, 11) & jnp.int32(2047)], ones_i)
            plsc.addupdate_scatter(
                histC, [lax.shift_right_logical(kk, 22) & jnp.int32(1023)], ones_i)
        return carry

    def make_permute_body(h, shift, bmask, from_f32):
        def body(w, stage, carry):
            def vb(j, _):
                if from_f32:
                    kk = _to_key(stage[pl.ds(j * 16, 16)])
                else:
                    kk = stage[pl.ds(j * 16, 16)]
                d = lax.shift_right_logical(kk, shift) & bmask
                cnt, last = plsc.scan_count(d)
                ofs = plsc.load_gather(h, [d])
                pos = ofs + cnt - 1
                plsc.store_scatter(dest, [pos], kk)
                plsc.addupdate_scatter(h, [d], cnt, mask=last)
                return 0
            lax.fori_loop(0, NVW, vb, 0, unroll=4)
            return carry
        return body

    def dot_body(w, stage, accs):
        @plsc.parallel_loop(0, NVW, step=1, unroll=4, carry=accs)
        def vb(j, accs2):
            na, ca, sa, cb = accs2
            kk = dest[pl.ds(w * WIN + j * 16, 16)]
            v = _from_key(kk)
            kv = stage[pl.ds(j * 16, 16)]
            y1 = kv * v - ca
            t1 = na + y1
            ca = (t1 - na) - y1
            y2 = v * v - cb
            t2 = sa + y2
            cb = (t2 - sa) - y2
            return (t1, ca, t2, cb)
        return vb

    def col_body(ci, _):
        c = wid * CPW + ci
        zero_hist(histA, 2048)
        zero_hist(histB, 2048)
        zero_hist(histC, 1024)
        xslice = lambda w: xT.at[c, pl.ds(w * WIN, WIN)]
        streamed(xslice, (sf0, sf1), hist_all_body, 0)
        excl_prefix(histA, 2048)
        excl_prefix(histB, 2048)
        excl_prefix(histC, 1024)
        streamed(xslice, (sf0, sf1),
                 make_permute_body(histA, 0, jnp.int32(2047), True), 0)
        pltpu.sync_copy(dest, s1.at[c])
        streamed(lambda w: s1.at[c, pl.ds(w * WIN, WIN)], (si0, si1),
                 make_permute_body(histB, 11, jnp.int32(2047), False), 0)
        pltpu.sync_copy(dest, s2.at[c])
        streamed(lambda w: s2.at[c, pl.ds(w * WIN, WIN)], (si0, si1),
                 make_permute_body(histC, 22, jnp.int32(1023), False), 0)
        zf = jnp.zeros((16,), jnp.float32)
        na, _, sa, _ = streamed(lambda w: kvec.at[pl.ds(w * WIN, WIN)],
                                (sk0, sk1), dot_body, (zf, zf, zf, zf))
        row_num[...] = jnp.full((16,), 0.0, jnp.float32) + jnp.sum(na)
        row_ss[...] = jnp.full((16,), 0.0, jnp.float32) + jnp.sum(sa)
        pltpu.sync_copy(row_num, num_out.at[c])
        pltpu.sync_copy(row_ss, ss_out.at[c])
        return 0

    lax.fori_loop(0, CPW, col_body, 0)


def kernel(x):
    eps = 1e-05
    n, d = x.shape
    k = lax.stop_gradient(_weights(n).astype(x.dtype))
    k_norm = jnp.linalg.norm(k)
    xT = x.T
    num_rows, ss_rows, _, _ = _sw_sc(xT, k)
    num = num_rows[:, 0]
    ss = ss_rows[:, 0]
    s_norm = jnp.sqrt(ss)
    cos = num / jnp.maximum(k_norm * s_norm, eps)
    return 1.0 - jnp.abs(cos)
